# baseline (speedup 1.0000x reference)
import functools
import math

import jax
import jax.numpy as jnp
from jax import lax
from jax.experimental import pallas as pl
from jax.experimental.pallas import tpu as pltpu
from jax.experimental.pallas import tpu_sc as plsc

N = 10000
E = 320000
H = 128
V = 100
K = 64
CUT_R = 5.0
BETA = (2.0 / K * (1.0 - math.exp(-CUT_R))) ** -2

NC = 2
NS = 16
NW = NC * NS
E_PER_W = E // NW
CHUNK = 80
E_CHUNKS = E_PER_W // CHUNK
N_PER_W = 400
N_WORKERS_N = N // N_PER_W
N_CHUNKS = N_PER_W // CHUNK

VPAD = 104
DPAD = 8


def _tables_body(nemb_ref, wn_ref, bn_ref, demb_ref, wd_ref, bd_ref,
                 ntab_ref, dtab_ref):
    ntab_ref[...] = jnp.maximum(
        jnp.dot(nemb_ref[...], wn_ref[...],
                preferred_element_type=jnp.float32) + bn_ref[...], 0.0)
    dtab_ref[...] = jnp.maximum(
        jnp.dot(demb_ref[...], wd_ref[...],
                preferred_element_type=jnp.float32) + bd_ref[...], 0.0)


def _rbf_body(feat_ref, mu_ref, w_ref, b_ref, out_ref):
    d = feat_ref[...]
    t = jnp.exp(-d)
    diff = t - mu_ref[...]
    rbf = jnp.exp((-BETA) * diff * diff)
    acc = jnp.dot(rbf, w_ref[...], preferred_element_type=jnp.float32)
    out_ref[...] = jnp.maximum(acc + b_ref[...], 0.0)


def _sc_body(feat_hbm, nidx_hbm, dtab_hbm, ntab_hbm,
             eh_emb_hbm, h_hbm,
             feat_v, idx_v, rows_v, sem):
    wid = lax.axis_index("s") * NC + lax.axis_index("c")
    ebase = wid * E_PER_W

    def edge_chunk(c, carry):
        off = ebase + c * CHUNK
        pltpu.sync_copy(feat_hbm.at[pl.ds(off, CHUNK)], feat_v)
        for i in range(CHUNK // 16):
            v = feat_v[pl.ds(i * 16, 16)]
            v = jnp.minimum(jnp.maximum(v, 1.0), 4.99999)
            idx_v[pl.ds(i * 16, 16)] = v.astype(jnp.int32) - 1
        pltpu.async_copy(dtab_hbm.at[idx_v], rows_v, sem).wait()
        pltpu.sync_copy(rows_v, eh_emb_hbm.at[pl.ds(off, CHUNK)])
        return carry

    lax.fori_loop(0, E_CHUNKS, edge_chunk, 0)

    @pl.when(wid < N_WORKERS_N)
    def _node_path():
        nbase = wid * N_PER_W

        def node_chunk(c, carry):
            off = nbase + c * CHUNK
            pltpu.sync_copy(nidx_hbm.at[pl.ds(off, CHUNK)], idx_v)
            pltpu.async_copy(ntab_hbm.at[idx_v], rows_v, sem).wait()
            pltpu.sync_copy(rows_v, h_hbm.at[pl.ds(off, CHUNK)])
            return carry

        lax.fori_loop(0, N_CHUNKS, node_chunk, 0)


def kernel(node_feat_continuous, node_feat_discrete, edge_feat_continuous,
           node_emb, dist_emb, W_node, b_node, W_dist, b_dist,
           W_edge, b_edge, mu):
    f32 = jnp.float32
    nemb_p = jnp.zeros((VPAD, H), f32).at[:V].set(node_emb)
    demb_p = jnp.zeros((DPAD, H), f32).at[:dist_emb.shape[0]].set(dist_emb)
    bn2 = b_node.reshape(1, H)
    bd2 = b_dist.reshape(1, H)
    be2 = b_edge.reshape(1, H)
    mu2 = mu.reshape(1, K)
    feat_flat = edge_feat_continuous.reshape(E)
    nidx = node_feat_discrete.astype(jnp.int32)

    ntab, dtab = pl.pallas_call(
        _tables_body,
        out_shape=(jax.ShapeDtypeStruct((VPAD, H), f32),
                   jax.ShapeDtypeStruct((DPAD, H), f32)),
    )(nemb_p, W_node, bn2, demb_p, W_dist, bd2)

    sc_fn = functools.partial(
        pl.kernel,
        mesh=plsc.VectorSubcoreMesh(core_axis_name="c", subcore_axis_name="s"),
        out_type=(jax.ShapeDtypeStruct((E, H), f32),
                  jax.ShapeDtypeStruct((N, H), f32)),
        scratch_types=[
            pltpu.VMEM((CHUNK,), f32),
            pltpu.VMEM((CHUNK,), jnp.int32),
            pltpu.VMEM((CHUNK, H), f32),
            pltpu.SemaphoreType.DMA,
        ],
    )(_sc_body)
    eh_emb, h = sc_fn(feat_flat, nidx, dtab, ntab)

    BE = 2000
    eh_rbf = pl.pallas_call(
        _rbf_body,
        grid=(E // BE,),
        in_specs=[
            pl.BlockSpec((BE, 1), lambda i: (i, 0)),
            pl.BlockSpec((1, K), lambda i: (0, 0)),
            pl.BlockSpec((K, H), lambda i: (0, 0)),
            pl.BlockSpec((1, H), lambda i: (0, 0)),
        ],
        out_specs=pl.BlockSpec((BE, H), lambda i: (i, 0)),
        out_shape=jax.ShapeDtypeStruct((E, H), f32),
    )(edge_feat_continuous, mu2, W_edge, be2)

    return (h, node_feat_continuous, eh_rbf, eh_emb)


# speedup vs baseline: 1.0009x; 1.0009x over previous
import functools
import math

import jax
import jax.numpy as jnp
from jax import lax
from jax.experimental import pallas as pl
from jax.experimental.pallas import tpu as pltpu
from jax.experimental.pallas import tpu_sc as plsc

N = 10000
E = 320000
H = 128
V = 100
K = 64
CUT_R = 5.0
BETA = (2.0 / K * (1.0 - math.exp(-CUT_R))) ** -2

NC = 2
NS = 16
NW = NC * NS
E_PER_W = E // NW
CHUNK = 80
E_CHUNKS = E_PER_W // CHUNK
N_PER_W = 400
N_WORKERS_N = N // N_PER_W
N_CHUNKS = N_PER_W // CHUNK

VPAD = 104
DPAD = 8


def _tables_body(nemb_ref, wn_ref, bn_ref, demb_ref, wd_ref, bd_ref,
                 ntab_ref, dtab_ref):
    ntab_ref[...] = jnp.maximum(
        jnp.dot(nemb_ref[...], wn_ref[...],
                preferred_element_type=jnp.float32) + bn_ref[...], 0.0)
    dtab_ref[...] = jnp.maximum(
        jnp.dot(demb_ref[...], wd_ref[...],
                preferred_element_type=jnp.float32) + bd_ref[...], 0.0)


def _rbf_body(feat_ref, mu_ref, w_ref, b_ref, out_ref):
    d = feat_ref[...]
    t = jnp.exp(-d)
    diff = t - mu_ref[...]
    rbf = jnp.exp((-BETA) * diff * diff)
    acc = jnp.dot(rbf, w_ref[...], preferred_element_type=jnp.float32)
    out_ref[...] = jnp.maximum(acc + b_ref[...], 0.0)


def _sc_body(feat_hbm, nidx_hbm, dtab_hbm, ntab_hbm,
             eh_emb_hbm, h_hbm,
             feat_v, idx_v, nidx_v, rows0, rows1, g0, g1, w0, w1):
    wid = lax.axis_index("s") * NC + lax.axis_index("c")
    ebase = wid * E_PER_W

    pltpu.sync_copy(feat_hbm.at[pl.ds(ebase, E_PER_W)], feat_v)

    def idx_block(j, carry):
        for i in range(25):
            o = (j * 25 + i) * 16
            v = feat_v[pl.ds(o, 16)]
            v = jnp.minimum(jnp.maximum(v, 1.0), 4.99999)
            idx_v[pl.ds(o, 16)] = v.astype(jnp.int32) - 1
        return carry

    lax.fori_loop(0, E_PER_W // (16 * 25), idx_block, 0)

    def g_start(c, buf, sem):
        pltpu.async_copy(dtab_hbm.at[idx_v.at[pl.ds(c * CHUNK, CHUNK)]],
                         buf, sem)

    def g_wait(buf, sem):
        pltpu.make_async_copy(eh_emb_hbm.at[pl.ds(0, CHUNK)], buf, sem).wait()

    def o_start(c, buf, sem):
        pltpu.async_copy(buf, eh_emb_hbm.at[pl.ds(ebase + c * CHUNK, CHUNK)],
                         sem)

    def o_wait(buf, sem):
        pltpu.make_async_copy(buf, eh_emb_hbm.at[pl.ds(0, CHUNK)], sem).wait()

    g_start(0, rows0, g0)
    g_wait(rows0, g0)
    o_start(0, rows0, w0)
    g_start(1, rows1, g1)

    def pair(p, carry):
        c = 2 * p + 1
        g_wait(rows1, g1)
        o_start(c, rows1, w1)
        o_wait(rows0, w0)
        g_start(c + 1, rows0, g0)
        g_wait(rows0, g0)
        o_start(c + 1, rows0, w0)
        o_wait(rows1, w1)
        g_start(c + 2, rows1, g1)
        return carry

    lax.fori_loop(0, (E_CHUNKS - 3) // 2, pair, 0)

    g_wait(rows1, g1)
    o_start(E_CHUNKS - 2, rows1, w1)
    o_wait(rows0, w0)
    g_start(E_CHUNKS - 1, rows0, g0)
    g_wait(rows0, g0)
    o_start(E_CHUNKS - 1, rows0, w0)
    o_wait(rows1, w1)
    o_wait(rows0, w0)

    @pl.when(wid < N_WORKERS_N)
    def _node_path():
        nbase = wid * N_PER_W
        pltpu.sync_copy(nidx_hbm.at[pl.ds(nbase, N_PER_W)], nidx_v)

        def n_fire(c):
            pltpu.async_copy(
                ntab_hbm.at[nidx_v.at[pl.ds(c * CHUNK, CHUNK)]],
                rows0 if c % 2 == 0 else rows1,
                g0 if c % 2 == 0 else g1)

        n_fire(0)
        for c in range(N_CHUNKS):
            if c + 1 < N_CHUNKS:
                n_fire(c + 1)
            buf = rows0 if c % 2 == 0 else rows1
            sem = g0 if c % 2 == 0 else g1
            pltpu.make_async_copy(h_hbm.at[pl.ds(0, CHUNK)], buf, sem).wait()
            pltpu.sync_copy(buf, h_hbm.at[pl.ds(nbase + c * CHUNK, CHUNK)])


def kernel(node_feat_continuous, node_feat_discrete, edge_feat_continuous,
           node_emb, dist_emb, W_node, b_node, W_dist, b_dist,
           W_edge, b_edge, mu):
    f32 = jnp.float32
    nemb_p = jnp.zeros((VPAD, H), f32).at[:V].set(node_emb)
    demb_p = jnp.zeros((DPAD, H), f32).at[:dist_emb.shape[0]].set(dist_emb)
    bn2 = b_node.reshape(1, H)
    bd2 = b_dist.reshape(1, H)
    be2 = b_edge.reshape(1, H)
    mu2 = mu.reshape(1, K)
    feat_flat = edge_feat_continuous.reshape(E)
    nidx = node_feat_discrete.astype(jnp.int32)

    ntab, dtab = pl.pallas_call(
        _tables_body,
        out_shape=(jax.ShapeDtypeStruct((VPAD, H), f32),
                   jax.ShapeDtypeStruct((DPAD, H), f32)),
    )(nemb_p, W_node, bn2, demb_p, W_dist, bd2)

    sc_fn = functools.partial(
        pl.kernel,
        mesh=plsc.VectorSubcoreMesh(core_axis_name="c", subcore_axis_name="s"),
        out_type=(jax.ShapeDtypeStruct((E, H), f32),
                  jax.ShapeDtypeStruct((N, H), f32)),
        scratch_types=[
            pltpu.VMEM((E_PER_W,), f32),
            pltpu.VMEM((E_PER_W,), jnp.int32),
            pltpu.VMEM((N_PER_W,), jnp.int32),
            pltpu.VMEM((CHUNK, H), f32),
            pltpu.VMEM((CHUNK, H), f32),
            pltpu.SemaphoreType.DMA,
            pltpu.SemaphoreType.DMA,
            pltpu.SemaphoreType.DMA,
            pltpu.SemaphoreType.DMA,
        ],
    )(_sc_body)
    eh_emb, h = sc_fn(feat_flat, nidx, dtab, ntab)

    BE = 2000
    eh_rbf = pl.pallas_call(
        _rbf_body,
        grid=(E // BE,),
        in_specs=[
            pl.BlockSpec((BE, 1), lambda i: (i, 0)),
            pl.BlockSpec((1, K), lambda i: (0, 0)),
            pl.BlockSpec((K, H), lambda i: (0, 0)),
            pl.BlockSpec((1, H), lambda i: (0, 0)),
        ],
        out_specs=pl.BlockSpec((BE, H), lambda i: (i, 0)),
        out_shape=jax.ShapeDtypeStruct((E, H), f32),
    )(edge_feat_continuous, mu2, W_edge, be2)

    return (h, node_feat_continuous, eh_rbf, eh_emb)


# speedup vs baseline: 8.0027x; 7.9955x over previous
import functools
import math

import jax
import jax.numpy as jnp
from jax import lax
from jax.experimental import pallas as pl
from jax.experimental.pallas import tpu as pltpu
from jax.experimental.pallas import tpu_sc as plsc

N = 10000
E = 320000
H = 128
V = 100
K = 64
CUT_R = 5.0
BETA = (2.0 / K * (1.0 - math.exp(-CUT_R))) ** -2

NC = 2
NS = 16
NW = NC * NS
E_PER_W = E // NW
CHUNK = 80
E_CHUNKS = E_PER_W // CHUNK
N_PER_W = 400
N_WORKERS_N = N // N_PER_W
N_CHUNKS = N_PER_W // CHUNK

VPAD = 104
DPAD = 8


def _tables_body(nemb_ref, wn_ref, bn_ref, demb_ref, wd_ref, bd_ref,
                 ntab_ref, dtab_ref):
    ntab_ref[...] = jnp.maximum(
        jnp.dot(nemb_ref[...], wn_ref[...],
                preferred_element_type=jnp.float32) + bn_ref[...], 0.0)
    dtab_ref[...] = jnp.maximum(
        jnp.dot(demb_ref[...], wd_ref[...],
                preferred_element_type=jnp.float32) + bd_ref[...], 0.0)


def _rbf_body(feat_ref, mu_ref, w_ref, b_ref, out_ref):
    d = feat_ref[...]
    t = jnp.exp(-d)
    diff = t - mu_ref[...]
    rbf = jnp.exp((-BETA) * diff * diff)
    acc = jnp.dot(rbf, w_ref[...], preferred_element_type=jnp.float32)
    out_ref[...] = jnp.maximum(acc + b_ref[...], 0.0)


def _sc_body(feat_hbm, nidx_hbm, dtab_hbm, ntab_hbm,
             eh_emb_hbm, h_hbm,
             feat_v, idx_v, nidx_v, rows0, rows1, g0, g1, w0, w1):
    wid = lax.axis_index("s") * NC + lax.axis_index("c")
    ebase = wid * E_PER_W

    pltpu.sync_copy(feat_hbm.at[pl.ds(ebase, E_PER_W)], feat_v)

    tab_base = wid * DPAD

    def idx_block(j, carry):
        for i in range(25):
            o = (j * 25 + i) * 16
            v = feat_v[pl.ds(o, 16)]
            v = jnp.minimum(jnp.maximum(v, 1.0), 4.99999)
            idx_v[pl.ds(o, 16)] = v.astype(jnp.int32) + (tab_base - 1)
        return carry

    lax.fori_loop(0, E_PER_W // (16 * 25), idx_block, 0)

    def g_start(c, buf, sem):
        pltpu.async_copy(dtab_hbm.at[idx_v.at[pl.ds(c * CHUNK, CHUNK)]],
                         buf, sem)

    def g_wait(buf, sem):
        pltpu.make_async_copy(eh_emb_hbm.at[pl.ds(0, CHUNK)], buf, sem).wait()

    def o_start(c, buf, sem):
        pltpu.async_copy(buf, eh_emb_hbm.at[pl.ds(ebase + c * CHUNK, CHUNK)],
                         sem)

    def o_wait(buf, sem):
        pltpu.make_async_copy(buf, eh_emb_hbm.at[pl.ds(0, CHUNK)], sem).wait()

    g_start(0, rows0, g0)
    g_wait(rows0, g0)
    o_start(0, rows0, w0)
    g_start(1, rows1, g1)

    def pair(p, carry):
        c = 2 * p + 1
        g_wait(rows1, g1)
        o_start(c, rows1, w1)
        o_wait(rows0, w0)
        g_start(c + 1, rows0, g0)
        g_wait(rows0, g0)
        o_start(c + 1, rows0, w0)
        o_wait(rows1, w1)
        g_start(c + 2, rows1, g1)
        return carry

    lax.fori_loop(0, (E_CHUNKS - 3) // 2, pair, 0)

    g_wait(rows1, g1)
    o_start(E_CHUNKS - 2, rows1, w1)
    o_wait(rows0, w0)
    g_start(E_CHUNKS - 1, rows0, g0)
    g_wait(rows0, g0)
    o_start(E_CHUNKS - 1, rows0, w0)
    o_wait(rows1, w1)
    o_wait(rows0, w0)

    @pl.when(wid < N_WORKERS_N)
    def _node_path():
        nbase = wid * N_PER_W
        pltpu.sync_copy(nidx_hbm.at[pl.ds(nbase, N_PER_W)], nidx_v)

        def n_fire(c):
            pltpu.async_copy(
                ntab_hbm.at[nidx_v.at[pl.ds(c * CHUNK, CHUNK)]],
                rows0 if c % 2 == 0 else rows1,
                g0 if c % 2 == 0 else g1)

        n_fire(0)
        for c in range(N_CHUNKS):
            if c + 1 < N_CHUNKS:
                n_fire(c + 1)
            buf = rows0 if c % 2 == 0 else rows1
            sem = g0 if c % 2 == 0 else g1
            pltpu.make_async_copy(h_hbm.at[pl.ds(0, CHUNK)], buf, sem).wait()
            pltpu.sync_copy(buf, h_hbm.at[pl.ds(nbase + c * CHUNK, CHUNK)])


def kernel(node_feat_continuous, node_feat_discrete, edge_feat_continuous,
           node_emb, dist_emb, W_node, b_node, W_dist, b_dist,
           W_edge, b_edge, mu):
    f32 = jnp.float32
    nemb_p = jnp.zeros((VPAD, H), f32).at[:V].set(node_emb)
    demb_p = jnp.zeros((DPAD, H), f32).at[:dist_emb.shape[0]].set(dist_emb)
    bn2 = b_node.reshape(1, H)
    bd2 = b_dist.reshape(1, H)
    be2 = b_edge.reshape(1, H)
    mu2 = mu.reshape(1, K)
    feat_flat = edge_feat_continuous.reshape(E)
    nidx = node_feat_discrete.astype(jnp.int32)

    ntab, dtab = pl.pallas_call(
        _tables_body,
        out_shape=(jax.ShapeDtypeStruct((VPAD, H), f32),
                   jax.ShapeDtypeStruct((DPAD, H), f32)),
    )(nemb_p, W_node, bn2, demb_p, W_dist, bd2)

    dtab_rep = jnp.tile(dtab, (NW, 1))

    sc_fn = functools.partial(
        pl.kernel,
        mesh=plsc.VectorSubcoreMesh(core_axis_name="c", subcore_axis_name="s"),
        out_type=(jax.ShapeDtypeStruct((E, H), f32),
                  jax.ShapeDtypeStruct((N, H), f32)),
        scratch_types=[
            pltpu.VMEM((E_PER_W,), f32),
            pltpu.VMEM((E_PER_W,), jnp.int32),
            pltpu.VMEM((N_PER_W,), jnp.int32),
            pltpu.VMEM((CHUNK, H), f32),
            pltpu.VMEM((CHUNK, H), f32),
            pltpu.SemaphoreType.DMA,
            pltpu.SemaphoreType.DMA,
            pltpu.SemaphoreType.DMA,
            pltpu.SemaphoreType.DMA,
        ],
    )(_sc_body)
    eh_emb, h = sc_fn(feat_flat, nidx, dtab_rep, ntab)

    BE = 2000
    eh_rbf = pl.pallas_call(
        _rbf_body,
        grid=(E // BE,),
        in_specs=[
            pl.BlockSpec((BE, 1), lambda i: (i, 0)),
            pl.BlockSpec((1, K), lambda i: (0, 0)),
            pl.BlockSpec((K, H), lambda i: (0, 0)),
            pl.BlockSpec((1, H), lambda i: (0, 0)),
        ],
        out_specs=pl.BlockSpec((BE, H), lambda i: (i, 0)),
        out_shape=jax.ShapeDtypeStruct((E, H), f32),
    )(edge_feat_continuous, mu2, W_edge, be2)

    return (h, node_feat_continuous, eh_rbf, eh_emb)


# speedup vs baseline: 16.2621x; 2.0321x over previous
import functools
import math

import jax
import jax.numpy as jnp
from jax import lax
from jax.experimental import pallas as pl
from jax.experimental.pallas import tpu as pltpu
from jax.experimental.pallas import tpu_sc as plsc

N = 10000
E = 320000
H = 128
V = 100
K = 64
CUT_R = 5.0
BETA = (2.0 / K * (1.0 - math.exp(-CUT_R))) ** -2

NC = 2
NS = 16
NW = NC * NS
E_PER_W = E // NW
CHUNK = 80
E_CHUNKS = E_PER_W // CHUNK
N_PER_W = 400
N_WORKERS_N = N // N_PER_W
N_CHUNKS = N_PER_W // CHUNK

VPAD = 104
DPAD = 8


def _tables_body(nemb_ref, wn_ref, bn_ref, demb_ref, wd_ref, bd_ref,
                 ntab_ref, dtab_ref):
    ntab_ref[...] = jnp.maximum(
        jnp.dot(nemb_ref[...], wn_ref[...],
                preferred_element_type=jnp.float32) + bn_ref[...], 0.0)
    dtab_ref[...] = jnp.maximum(
        jnp.dot(demb_ref[...], wd_ref[...],
                preferred_element_type=jnp.float32) + bd_ref[...], 0.0)


def _rbf_body(feat_ref, mu_ref, w_ref, b_ref, out_ref):
    d = feat_ref[...]
    t = jnp.exp(-d)
    diff = t - mu_ref[...]
    rbf = jnp.exp((-BETA) * diff * diff)
    acc = jnp.dot(rbf, w_ref[...], preferred_element_type=jnp.float32)
    out_ref[...] = jnp.maximum(acc + b_ref[...], 0.0)


def _sc_body(feat_hbm, nidx_hbm, dtab_hbm, ntab_hbm,
             eh_emb_hbm, h_hbm,
             feat_v, idx_v, nidx_v, tab_v, rows0, rows1, g0, g1, w0, w1):
    wid = lax.axis_index("s") * NC + lax.axis_index("c")
    ebase = wid * E_PER_W

    @pl.when(lax.axis_index("s") == 0)
    def _stage_table():
        pltpu.sync_copy(dtab_hbm.at[pl.ds(0, DPAD)], tab_v)

    plsc.subcore_barrier()

    pltpu.sync_copy(feat_hbm.at[pl.ds(ebase, E_PER_W)], feat_v)

    tab_base = 0

    def idx_block(j, carry):
        for i in range(25):
            o = (j * 25 + i) * 16
            v = feat_v[pl.ds(o, 16)]
            v = jnp.minimum(jnp.maximum(v, 1.0), 4.99999)
            idx_v[pl.ds(o, 16)] = v.astype(jnp.int32) + (tab_base - 1)
        return carry

    lax.fori_loop(0, E_PER_W // (16 * 25), idx_block, 0)

    def g_start(c, buf, sem):
        pltpu.async_copy(tab_v.at[idx_v.at[pl.ds(c * CHUNK, CHUNK)]],
                         buf, sem)

    def g_wait(buf, sem):
        pltpu.make_async_copy(eh_emb_hbm.at[pl.ds(0, CHUNK)], buf, sem).wait()

    def o_start(c, buf, sem):
        pltpu.async_copy(buf, eh_emb_hbm.at[pl.ds(ebase + c * CHUNK, CHUNK)],
                         sem)

    def o_wait(buf, sem):
        pltpu.make_async_copy(buf, eh_emb_hbm.at[pl.ds(0, CHUNK)], sem).wait()

    g_start(0, rows0, g0)
    g_wait(rows0, g0)
    o_start(0, rows0, w0)
    g_start(1, rows1, g1)

    def pair(p, carry):
        c = 2 * p + 1
        g_wait(rows1, g1)
        o_start(c, rows1, w1)
        o_wait(rows0, w0)
        g_start(c + 1, rows0, g0)
        g_wait(rows0, g0)
        o_start(c + 1, rows0, w0)
        o_wait(rows1, w1)
        g_start(c + 2, rows1, g1)
        return carry

    lax.fori_loop(0, (E_CHUNKS - 3) // 2, pair, 0)

    g_wait(rows1, g1)
    o_start(E_CHUNKS - 2, rows1, w1)
    o_wait(rows0, w0)
    g_start(E_CHUNKS - 1, rows0, g0)
    g_wait(rows0, g0)
    o_start(E_CHUNKS - 1, rows0, w0)
    o_wait(rows1, w1)
    o_wait(rows0, w0)

    @pl.when(wid < N_WORKERS_N)
    def _node_path():
        nbase = wid * N_PER_W
        pltpu.sync_copy(nidx_hbm.at[pl.ds(nbase, N_PER_W)], nidx_v)

        def n_fire(c):
            pltpu.async_copy(
                ntab_hbm.at[nidx_v.at[pl.ds(c * CHUNK, CHUNK)]],
                rows0 if c % 2 == 0 else rows1,
                g0 if c % 2 == 0 else g1)

        n_fire(0)
        for c in range(N_CHUNKS):
            if c + 1 < N_CHUNKS:
                n_fire(c + 1)
            buf = rows0 if c % 2 == 0 else rows1
            sem = g0 if c % 2 == 0 else g1
            pltpu.make_async_copy(h_hbm.at[pl.ds(0, CHUNK)], buf, sem).wait()
            pltpu.sync_copy(buf, h_hbm.at[pl.ds(nbase + c * CHUNK, CHUNK)])


def kernel(node_feat_continuous, node_feat_discrete, edge_feat_continuous,
           node_emb, dist_emb, W_node, b_node, W_dist, b_dist,
           W_edge, b_edge, mu):
    f32 = jnp.float32
    nemb_p = jnp.zeros((VPAD, H), f32).at[:V].set(node_emb)
    demb_p = jnp.zeros((DPAD, H), f32).at[:dist_emb.shape[0]].set(dist_emb)
    bn2 = b_node.reshape(1, H)
    bd2 = b_dist.reshape(1, H)
    be2 = b_edge.reshape(1, H)
    mu2 = mu.reshape(1, K)
    feat_flat = edge_feat_continuous.reshape(E)
    nidx = node_feat_discrete.astype(jnp.int32)

    ntab, dtab = pl.pallas_call(
        _tables_body,
        out_shape=(jax.ShapeDtypeStruct((VPAD, H), f32),
                   jax.ShapeDtypeStruct((DPAD, H), f32)),
    )(nemb_p, W_node, bn2, demb_p, W_dist, bd2)

    dtab_rep = jnp.tile(dtab, (NW, 1))

    sc_fn = functools.partial(
        pl.kernel,
        mesh=plsc.VectorSubcoreMesh(core_axis_name="c", subcore_axis_name="s"),
        out_type=(jax.ShapeDtypeStruct((E, H), f32),
                  jax.ShapeDtypeStruct((N, H), f32)),
        scratch_types=[
            pltpu.VMEM((E_PER_W,), f32),
            pltpu.VMEM((E_PER_W,), jnp.int32),
            pltpu.VMEM((N_PER_W,), jnp.int32),
            pltpu.VMEM_SHARED((DPAD, H), f32),
            pltpu.VMEM((CHUNK, H), f32),
            pltpu.VMEM((CHUNK, H), f32),
            pltpu.SemaphoreType.DMA,
            pltpu.SemaphoreType.DMA,
            pltpu.SemaphoreType.DMA,
            pltpu.SemaphoreType.DMA,
        ],
    )(_sc_body)
    eh_emb, h = sc_fn(feat_flat, nidx, dtab_rep, ntab)

    BE = 2000
    eh_rbf = pl.pallas_call(
        _rbf_body,
        grid=(E // BE,),
        in_specs=[
            pl.BlockSpec((BE, 1), lambda i: (i, 0)),
            pl.BlockSpec((1, K), lambda i: (0, 0)),
            pl.BlockSpec((K, H), lambda i: (0, 0)),
            pl.BlockSpec((1, H), lambda i: (0, 0)),
        ],
        out_specs=pl.BlockSpec((BE, H), lambda i: (i, 0)),
        out_shape=jax.ShapeDtypeStruct((E, H), f32),
    )(edge_feat_continuous, mu2, W_edge, be2)

    return (h, node_feat_continuous, eh_rbf, eh_emb)


# speedup vs baseline: 16.3856x; 1.0076x over previous
import functools
import math

import jax
import jax.numpy as jnp
from jax import lax
from jax.experimental import pallas as pl
from jax.experimental.pallas import tpu as pltpu
from jax.experimental.pallas import tpu_sc as plsc

N = 10000
E = 320000
H = 128
V = 100
K = 64
CUT_R = 5.0
BETA = (2.0 / K * (1.0 - math.exp(-CUT_R))) ** -2
NDIST = int(CUT_R) - 1

NC = 2
NS = 16
NW = NC * NS
E_PER_W = E // NW
CHUNK = 80
E_CHUNKS = E_PER_W // CHUNK
N_PER_W = 400
N_WORKERS_N = N // N_PER_W
N_CHUNKS = N_PER_W // CHUNK


def _tables_body(nemb_ref, wn_ref, bn_ref, demb_ref, wd_ref, bd_ref,
                 ntab_ref, dtab_ref):
    ntab_ref[...] = jnp.maximum(
        jnp.dot(nemb_ref[...], wn_ref[...],
                preferred_element_type=jnp.float32) + bn_ref[...], 0.0)
    dtab_ref[...] = jnp.maximum(
        jnp.dot(demb_ref[...], wd_ref[...],
                preferred_element_type=jnp.float32) + bd_ref[...], 0.0)


def _rbf_body(feat_ref, mu_ref, w_ref, b_ref, out_ref):
    d = feat_ref[...]
    t = jnp.exp(-d)
    diff = t - mu_ref[...]
    rbf = jnp.exp((-BETA) * diff * diff)
    acc = jnp.dot(rbf, w_ref[...], preferred_element_type=jnp.float32)
    out_ref[...] = jnp.maximum(acc + b_ref[...], 0.0)


def _sc_body(feat_hbm, nidx_hbm, dtab_hbm, ntab_hbm, nfc_hbm,
             eh_emb_hbm, h_hbm, hin_hbm,
             feat_v, idx_v, nidx_v, tab_v, rows0, rows1, g0, g1, w0, w1):
    wid = lax.axis_index("s") * NC + lax.axis_index("c")
    ebase = wid * E_PER_W

    @pl.when(lax.axis_index("s") == 0)
    def _stage_table():
        pltpu.sync_copy(dtab_hbm, tab_v)

    plsc.subcore_barrier()

    pltpu.sync_copy(feat_hbm.at[pl.ds(ebase, E_PER_W)], feat_v)

    def idx_block(j, carry):
        for i in range(25):
            o = (j * 25 + i) * 16
            v = feat_v[pl.ds(o, 16)]
            v = jnp.minimum(jnp.maximum(v, 1.0), 4.99999)
            idx_v[pl.ds(o, 16)] = v.astype(jnp.int32) - 1
        return carry

    lax.fori_loop(0, E_PER_W // (16 * 25), idx_block, 0)

    def g_start(c, buf, sem):
        pltpu.async_copy(tab_v.at[idx_v.at[pl.ds(c * CHUNK, CHUNK)]],
                         buf, sem)

    def g_wait(buf, sem):
        pltpu.make_async_copy(eh_emb_hbm.at[pl.ds(0, CHUNK)], buf, sem).wait()

    def o_start(c, buf, sem):
        pltpu.async_copy(buf, eh_emb_hbm.at[pl.ds(ebase + c * CHUNK, CHUNK)],
                         sem)

    def o_wait(buf, sem):
        pltpu.make_async_copy(buf, eh_emb_hbm.at[pl.ds(0, CHUNK)], sem).wait()

    g_start(0, rows0, g0)
    g_wait(rows0, g0)
    o_start(0, rows0, w0)
    g_start(1, rows1, g1)

    def pair(p, carry):
        c = 2 * p + 1
        g_wait(rows1, g1)
        o_start(c, rows1, w1)
        o_wait(rows0, w0)
        g_start(c + 1, rows0, g0)
        g_wait(rows0, g0)
        o_start(c + 1, rows0, w0)
        o_wait(rows1, w1)
        g_start(c + 2, rows1, g1)
        return carry

    lax.fori_loop(0, (E_CHUNKS - 3) // 2, pair, 0)

    g_wait(rows1, g1)
    o_start(E_CHUNKS - 2, rows1, w1)
    o_wait(rows0, w0)
    g_start(E_CHUNKS - 1, rows0, g0)
    g_wait(rows0, g0)
    o_start(E_CHUNKS - 1, rows0, w0)
    o_wait(rows1, w1)
    o_wait(rows0, w0)

    @pl.when(wid < N_WORKERS_N)
    def _node_path():
        nbase = wid * N_PER_W
        pltpu.sync_copy(nidx_hbm.at[pl.ds(nbase, N_PER_W)], nidx_v)
        for c in range(N_CHUNKS):
            off = nbase + c * CHUNK
            pltpu.async_copy(nfc_hbm.at[pl.ds(off, CHUNK)], rows1, g1)
            pltpu.async_copy(
                ntab_hbm.at[nidx_v.at[pl.ds(c * CHUNK, CHUNK)]], rows0, g0)
            pltpu.make_async_copy(nfc_hbm.at[pl.ds(0, CHUNK)], rows1,
                                  g1).wait()
            pltpu.sync_copy(rows1, hin_hbm.at[pl.ds(off, CHUNK)])
            pltpu.make_async_copy(h_hbm.at[pl.ds(0, CHUNK)], rows0, g0).wait()
            pltpu.sync_copy(rows0, h_hbm.at[pl.ds(off, CHUNK)])


def kernel(node_feat_continuous, node_feat_discrete, edge_feat_continuous,
           node_emb, dist_emb, W_node, b_node, W_dist, b_dist,
           W_edge, b_edge, mu):
    f32 = jnp.float32
    bn2 = b_node.reshape(1, H)
    bd2 = b_dist.reshape(1, H)
    be2 = b_edge.reshape(1, H)
    mu2 = mu.reshape(1, K)
    feat_flat = edge_feat_continuous.reshape(E)
    nidx = node_feat_discrete.astype(jnp.int32)

    ntab, dtab = pl.pallas_call(
        _tables_body,
        out_shape=(jax.ShapeDtypeStruct((V, H), f32),
                   jax.ShapeDtypeStruct((NDIST, H), f32)),
    )(node_emb, W_node, bn2, dist_emb, W_dist, bd2)

    sc_fn = functools.partial(
        pl.kernel,
        mesh=plsc.VectorSubcoreMesh(core_axis_name="c", subcore_axis_name="s"),
        out_type=(jax.ShapeDtypeStruct((E, H), f32),
                  jax.ShapeDtypeStruct((N, H), f32),
                  jax.ShapeDtypeStruct((N, H), f32)),
        scratch_types=[
            pltpu.VMEM((E_PER_W,), f32),
            pltpu.VMEM((E_PER_W,), jnp.int32),
            pltpu.VMEM((N_PER_W,), jnp.int32),
            pltpu.VMEM_SHARED((NDIST, H), f32),
            pltpu.VMEM((CHUNK, H), f32),
            pltpu.VMEM((CHUNK, H), f32),
            pltpu.SemaphoreType.DMA,
            pltpu.SemaphoreType.DMA,
            pltpu.SemaphoreType.DMA,
            pltpu.SemaphoreType.DMA,
        ],
    )(_sc_body)
    eh_emb, h, h_in = sc_fn(feat_flat, nidx, dtab, ntab, node_feat_continuous)

    BE = 2000
    eh_rbf = pl.pallas_call(
        _rbf_body,
        grid=(E // BE,),
        in_specs=[
            pl.BlockSpec((BE, 1), lambda i: (i, 0)),
            pl.BlockSpec((1, K), lambda i: (0, 0)),
            pl.BlockSpec((K, H), lambda i: (0, 0)),
            pl.BlockSpec((1, H), lambda i: (0, 0)),
        ],
        out_specs=pl.BlockSpec((BE, H), lambda i: (i, 0)),
        out_shape=jax.ShapeDtypeStruct((E, H), f32),
    )(edge_feat_continuous, mu2, W_edge, be2)

    return (h, h_in, eh_rbf, eh_emb)
